# Initial kernel scaffold; baseline (speedup 1.0000x reference)
import jax
import jax.numpy as jnp
from jax.experimental import pallas as pl


def kernel(logits, target_codon_ids, aa_ids, species_ids, mask, weight_matrix, ref_distributions):
    raise NotImplementedError("write your pallas kernel here")


# speedup vs baseline: 11.9664x; 11.9664x over previous
import numpy as np
import jax
import jax.numpy as jnp
from jax.experimental import pallas as pl
from jax.experimental.pallas import tpu as pltpu

N_CODONS = 66
N_SPECIES = 5
B, L = 64, 4096
TL = 512
NT = L // TL

_AA_TABLE = "FFLLSSSSYY**CC*WLLLLPPPPHHQQRRRRIIIMTTTTNNKKSSRRVVVVAAAADDEEGGGG"
_letters = sorted(set(_AA_TABLE))
_GRP = {a: i for i, a in enumerate(_letters)}
N_GROUPS = len(_letters)
_c2g = np.full((N_CODONS,), N_GROUPS, dtype=np.int32)
for _i, _a in enumerate(_AA_TABLE):
    _c2g[_i + 1] = _GRP[_a]
_nsg = np.zeros((N_GROUPS + 1,), dtype=np.float32)
for _a in _AA_TABLE:
    _nsg[_GRP[_a]] += 1.0
_nsg[N_GROUPS] = 1.0
_NSYN = _nsg[_c2g]
_SAME = (_c2g[:, None] == _c2g[None, :]).astype(np.float32)
_KEEP = ((np.arange(N_CODONS) > 0) & (np.arange(N_CODONS) < 65)).astype(np.float32)

_NSYN_J = jnp.asarray(_NSYN[None, :])
_SAME_J = jnp.asarray(_SAME)
_KEEP_J = jnp.asarray(_KEEP[None, :])


def _loss_kernel(sp1h_ref, wm_ref, refd_ref, same_ref, nsyn_ref, keep_ref,
                 logits_ref, tgt_ref, mask_ref,
                 out_ref, histp_ref, histt_ref, misc_ref):
    b = pl.program_id(0)
    t = pl.program_id(1)

    @pl.when((b == 0) & (t == 0))
    def _init():
        histp_ref[...] = jnp.zeros_like(histp_ref)
        histt_ref[...] = jnp.zeros_like(histt_ref)
        misc_ref[0] = 0.0
        misc_ref[1] = 0.0

    x = logits_ref[0]
    tgt = tgt_ref[0, 0]
    msk = mask_ref[0, 0]

    iota = jax.lax.broadcasted_iota(jnp.int32, (TL, N_CODONS), 1)
    m = jnp.max(x, axis=1, keepdims=True)
    sumexp = jnp.sum(jnp.exp(x - m), axis=1)
    lse = m[:, 0] + jnp.log(sumexp)
    tgt_oh = tgt[:, None] == iota
    tgt_logit = jnp.sum(jnp.where(tgt_oh, x, 0.0), axis=1)
    ce_m = (tgt != 0).astype(jnp.float32)
    misc_ref[0] += jnp.sum((lse - tgt_logit) * ce_m)
    misc_ref[1] += jnp.sum(ce_m)

    pred = jnp.min(jnp.where(x == m, iota, N_CODONS), axis=1)
    pred_oh = (pred[:, None] == iota).astype(jnp.float32) * msk[:, None]
    tgt_ohf = tgt_oh.astype(jnp.float32) * msk[:, None]
    histp_ref[pl.ds(b, 1), :] += jnp.sum(pred_oh, axis=0)[None, :]
    histt_ref[pl.ds(b, 1), :] += jnp.sum(tgt_ohf, axis=0)[None, :]

    @pl.when((b == B - 1) & (t == NT - 1))
    def _final():
        hp = histp_ref[...]
        ht = histt_ref[...]
        sp = sp1h_ref[...]
        lw = jnp.log(jnp.clip(wm_ref[...], 1e-8, None))
        lw_sel = jnp.dot(sp, lw, preferred_element_type=jnp.float32)
        cnt = jnp.clip(jnp.sum(ht, axis=1), 1.0, None)
        cai_p = jnp.exp(jnp.sum(hp * lw_sel, axis=1) / cnt)
        cai_t = jnp.exp(jnp.sum(ht * lw_sel, axis=1) / cnt)
        cai_loss = jnp.mean(jnp.maximum(cai_t - cai_p, 0.0))

        keep = keep_ref[...]
        nsyn = nsyn_ref[...]
        same = same_ref[...]

        def rscu(h):
            c = h * keep
            tot = jnp.dot(c, same, preferred_element_type=jnp.float32)
            return jnp.where(tot > 0, c * nsyn / jnp.maximum(tot, 1e-8), 0.0)

        rp = rscu(hp)
        rt = rscu(ht)
        rr = jnp.dot(sp, refd_ref[...], preferred_element_type=jnp.float32)
        q = 0.7 * rt + 0.3 * rr + 1e-8
        p = rp + 1e-8
        pd = p / jnp.sum(p, axis=1, keepdims=True)
        qd = q / jnp.sum(q, axis=1, keepdims=True)
        kl = jnp.sum(qd * jnp.log(qd / pd), axis=1)
        rscu_loss = jnp.mean(kl)
        ce = misc_ref[0] / jnp.maximum(misc_ref[1], 1.0)
        total = ce + 0.4 * cai_loss + 0.3 * rscu_loss
        out_ref[...] = jnp.full((1, 1), total, dtype=jnp.float32)


def kernel(logits, target_codon_ids, aa_ids, species_ids, mask,
           weight_matrix, ref_distributions):
    del aa_ids
    sp1h = jax.nn.one_hot(species_ids, N_SPECIES, dtype=jnp.float32)
    tgt3 = target_codon_ids.astype(jnp.int32).reshape(B, 1, L)
    mask3 = mask.astype(jnp.float32).reshape(B, 1, L)

    out = pl.pallas_call(
        _loss_kernel,
        grid=(B, NT),
        in_specs=[
            pl.BlockSpec((B, N_SPECIES), lambda b, t: (0, 0)),
            pl.BlockSpec((N_SPECIES, N_CODONS), lambda b, t: (0, 0)),
            pl.BlockSpec((N_SPECIES, N_CODONS), lambda b, t: (0, 0)),
            pl.BlockSpec((N_CODONS, N_CODONS), lambda b, t: (0, 0)),
            pl.BlockSpec((1, N_CODONS), lambda b, t: (0, 0)),
            pl.BlockSpec((1, N_CODONS), lambda b, t: (0, 0)),
            pl.BlockSpec((1, TL, N_CODONS), lambda b, t: (b, t, 0)),
            pl.BlockSpec((1, 1, TL), lambda b, t: (b, 0, t)),
            pl.BlockSpec((1, 1, TL), lambda b, t: (b, 0, t)),
        ],
        out_specs=pl.BlockSpec((1, 1), lambda b, t: (0, 0)),
        out_shape=jax.ShapeDtypeStruct((1, 1), jnp.float32),
        scratch_shapes=[
            pltpu.VMEM((B, N_CODONS), jnp.float32),
            pltpu.VMEM((B, N_CODONS), jnp.float32),
            pltpu.SMEM((2,), jnp.float32),
        ],
    )(sp1h, weight_matrix, ref_distributions, _SAME_J, _NSYN_J, _KEEP_J,
      logits, tgt3, mask3)
    return out[0, 0]
